# baseline (speedup 1.0000x reference)
import jax
import jax.numpy as jnp
from jax import lax
from jax.experimental import pallas as pl
from jax.experimental.pallas import tpu as pltpu
from jax.experimental.pallas import tpu_sc as plsc

NC = 2
NS = 16
L = 16
NW = NC * NS
B = 16384
D = 64
BPW = B // NW
CH = 128
NCH = BPW // CH
NG = CH // L


def _sqrt16(x):
    bits = plsc.bitcast(x, jnp.int32)
    y = plsc.bitcast(jnp.int32(0x1FBD1DF5) + (bits >> 1), jnp.float32)
    for _ in range(3):
        y = 0.5 * (y + x / y)
    return y


NE = 1000000
NBLK = NE // CH
NMAIN = (NBLK // NW) * NW
BPT = NBLK // NW
NTAIL = NE - NBLK * CH


def _transpose_body(entT, out, in0, in1, ov0, ov1, tin,
                    si0, si1, so0, so1):
    c = lax.axis_index("c")
    s = lax.axis_index("s")
    wid = s * NC + c
    blo = BPT * wid

    inb = (in0, in1)
    ovb = (ov0, ov1)
    sin = (si0, si1)
    sout = (so0, so1)
    iota = lax.iota(jnp.int32, L)

    def fire_in(b, slot):
        pltpu.make_async_copy(
            entT.at[:, pl.ds(b * CH, CH)], inb[slot], sin[slot]).start()

    def wait_in(slot):
        pltpu.make_async_copy(
            entT.at[:, pl.ds(0, CH)], inb[slot], sin[slot]).wait()

    def start_out(b, slot):
        pltpu.make_async_copy(
            ovb[slot], out.at[pl.ds(b * CH, CH)], sout[slot]).start()

    def wait_out(slot):
        pltpu.make_async_copy(
            ovb[slot], out.at[pl.ds(0, CH)], sout[slot]).wait()

    def transpose_into(buf, src, n):
        def row_t(e, carry):
            col = jnp.full((L,), 0, jnp.int32) + e
            for k in range(D // L):
                buf[e, pl.ds(k * L, L)] = plsc.load_gather(
                    src, [k * L + iota, col])
            return carry

        lax.fori_loop(0, n, row_t, 0, unroll=4)

    fire_in(blo, 0)
    fire_in(blo + 1, 1)

    wait_in(0)
    transpose_into(ov0, in0, CH)
    start_out(blo, 0)
    fire_in(blo + 2, 0)
    wait_in(1)
    transpose_into(ov1, in1, CH)
    start_out(blo + 1, 1)
    fire_in(blo + 3, 1)

    def pair(i, carry):
        b = blo + 2 * i
        for slot in range(2):
            wait_in(slot)
            wait_out(slot)
            transpose_into(ovb[slot], inb[slot], CH)
            start_out(b + slot, slot)
            fire_in(b + slot + 2, slot)
        return carry

    lax.fori_loop(1, BPT // 2 - 1, pair, 0)

    b = blo + BPT - 2
    for slot in range(2):
        wait_in(slot)
        wait_out(slot)
        transpose_into(ovb[slot], inb[slot], CH)
        start_out(b + slot, slot)
    wait_out(0)
    wait_out(1)

    @pl.when(wid < NBLK - NMAIN)
    def _():
        bb = NMAIN + wid
        pltpu.sync_copy(entT.at[:, pl.ds(bb * CH, CH)], in0)
        transpose_into(ov0, in0, CH)
        pltpu.sync_copy(ov0, out.at[pl.ds(bb * CH, CH)])

    @pl.when(wid == NW - 1)
    def _():
        pltpu.sync_copy(entT.at[:, pl.ds(NBLK * CH, NTAIL)], tin)

        def row_tail(e, carry):
            col = jnp.full((L,), 0, jnp.int32) + e
            for k in range(D // L):
                ov0[e, pl.ds(k * L, L)] = plsc.load_gather(
                    tin, [k * L + iota, col])
            return carry

        lax.fori_loop(0, NTAIL, row_tail, 0)
        pltpu.sync_copy(ov0.at[pl.ds(0, NTAIL)],
                        out.at[pl.ds(NBLK * CH, NTAIL)])


def _sc_body(h2d, r2d, t2d, ent, rel, out,
             hidx, ridx, tidx,
             hv0, rv0, tv0, hv1, rv1, tv1, ps, sc2,
             sh0, sr0, st0, sh1, sr1, st1):
    c = lax.axis_index("c")
    s = lax.axis_index("s")
    wid = s * NC + c

    pltpu.sync_copy(h2d.at[pl.ds(NCH * wid, NCH)], hidx)
    pltpu.sync_copy(r2d.at[pl.ds(NCH * wid, NCH)], ridx)
    pltpu.sync_copy(t2d.at[pl.ds(NCH * wid, NCH)], tidx)

    hv = (hv0, hv1)
    rv = (rv0, rv1)
    tv = (tv0, tv1)
    sems = ((sh0, sr0, st0), (sh1, sr1, st1))

    iota = lax.iota(jnp.int32, L)

    def fire(j):
        slot = j % 2

        def group_dma(g, carry, j=j, slot=slot):
            hvec = hidx[j, pl.ds(g * L, L)]
            rvec = ridx[j, pl.ds(g * L, L)]
            tvec = tidx[j, pl.ds(g * L, L)]
            for lane in range(L):
                m = iota == lane
                hid = lax.reduce_max(jnp.where(m, hvec, -1), axes=(0,))
                rid = lax.reduce_max(jnp.where(m, rvec, -1), axes=(0,))
                tid = lax.reduce_max(jnp.where(m, tvec, -1), axes=(0,))
                r = g * L + lane
                pltpu.make_async_copy(
                    ent.at[pl.ds(hid, 1)], hv[slot].at[pl.ds(r, 1)],
                    sems[slot][0]).start()
                pltpu.make_async_copy(
                    rel.at[pl.ds(rid, 1)], rv[slot].at[pl.ds(r, 1)],
                    sems[slot][1]).start()
                pltpu.make_async_copy(
                    ent.at[pl.ds(tid, 1)], tv[slot].at[pl.ds(r, 1)],
                    sems[slot][2]).start()
            return carry

        lax.fori_loop(0, NG, group_dma, 0)

    def drain(j):
        slot = j % 2
        pltpu.make_async_copy(
            ent.at[pl.ds(0, CH)], hv[slot], sems[slot][0]).wait()
        pltpu.make_async_copy(
            ent.at[pl.ds(0, CH)], rv[slot], sems[slot][1]).wait()
        pltpu.make_async_copy(
            ent.at[pl.ds(0, CH)], tv[slot], sems[slot][2]).wait()

    fire(0)
    fire(1)

    for j in range(NCH):
        slot = j % 2
        drain(j)

        def row_body(r, carry, slot=slot):
            acc = None
            for k in range(D // L):
                hh = hv[slot][r, pl.ds(k * L, L)]
                re = rv[slot][r, pl.ds(k * L, L)]
                tt = tv[slot][r, pl.ds(k * L, L)]
                df = (hh + re) - tt
                acc = df * df if acc is None else acc + df * df
            ps[r] = acc
            return carry

        lax.fori_loop(0, CH, row_body, 0, unroll=4)

        for g in range(NG):
            rows16 = g * L + iota
            acc = jnp.zeros((L,), jnp.float32)
            for k in range(L):
                col = jnp.full((L,), k, jnp.int32)
                acc = acc + plsc.load_gather(ps, [rows16, col])
            sc2[j, pl.ds(g * L, L)] = _sqrt16(acc)

        if j + 2 < NCH:
            fire(j + 2)

    pltpu.sync_copy(sc2, out.at[pl.ds(NCH * wid, NCH)])


@jax.jit
def kernel(head_ids, relation_ids, tail_ids, entity_table, relation_table):
    h2d = head_ids.astype(jnp.int32).reshape(NW * NCH, CH)
    r2d = relation_ids.astype(jnp.int32).reshape(NW * NCH, CH)
    t2d = tail_ids.astype(jnp.int32).reshape(NW * NCH, CH)

    mesh = plsc.VectorSubcoreMesh(core_axis_name="c", subcore_axis_name="s")

    entT = entity_table.T
    trans = pl.kernel(
        _transpose_body,
        out_type=jax.ShapeDtypeStruct((NE, D), jnp.float32),
        mesh=mesh,
        scratch_types=[
            pltpu.VMEM((D, CH), jnp.float32),
            pltpu.VMEM((D, CH), jnp.float32),
            pltpu.VMEM((CH, D), jnp.float32),
            pltpu.VMEM((CH, D), jnp.float32),
            pltpu.VMEM((D, NTAIL), jnp.float32),
        ] + [pltpu.SemaphoreType.DMA] * 4,
        compiler_params=pltpu.CompilerParams(
            needs_layout_passes=False, use_tc_tiling_on_sc=True),
    )
    ent_rm = trans(entT)

    scratch = [
        pltpu.VMEM((NCH, CH), jnp.int32),
        pltpu.VMEM((NCH, CH), jnp.int32),
        pltpu.VMEM((NCH, CH), jnp.int32),
        pltpu.VMEM((CH, D), jnp.float32),
        pltpu.VMEM((CH, D), jnp.float32),
        pltpu.VMEM((CH, D), jnp.float32),
        pltpu.VMEM((CH, D), jnp.float32),
        pltpu.VMEM((CH, D), jnp.float32),
        pltpu.VMEM((CH, D), jnp.float32),
        pltpu.VMEM((CH, L), jnp.float32),
        pltpu.VMEM((NCH, CH), jnp.float32),
    ] + [pltpu.SemaphoreType.DMA] * 6

    run = pl.kernel(
        _sc_body,
        out_type=jax.ShapeDtypeStruct((NW * NCH, CH), jnp.float32),
        mesh=mesh,
        scratch_types=scratch,
        compiler_params=pltpu.CompilerParams(
            needs_layout_passes=False, use_tc_tiling_on_sc=True),
    )
    out = run(h2d, r2d, t2d, ent_rm, relation_table)
    return out.reshape(B)


# speedup vs baseline: 4.0886x; 4.0886x over previous
import jax
import jax.numpy as jnp
from jax import lax
from jax.experimental import pallas as pl
from jax.experimental.pallas import tpu as pltpu
from jax.experimental.pallas import tpu_sc as plsc

NC = 2
NS = 16
L = 16
NW = NC * NS
B = 16384
D = 64
BPW = B // NW
CH = 128
NCH = BPW // CH
NG = CH // L


def _sqrt16(x):
    bits = plsc.bitcast(x, jnp.int32)
    y = plsc.bitcast(jnp.int32(0x1FBD1DF5) + (bits >> 1), jnp.float32)
    for _ in range(3):
        y = 0.5 * (y + x / y)
    return y


def _sc_body(h2d, r2d, t2d, ent, rel, out,
             hidx, ridx, tidx,
             hv0, rv0, tv0, hv1, rv1, tv1, ps, sc2,
             sh0, sr0, st0, sh1, sr1, st1):
    c = lax.axis_index("c")
    s = lax.axis_index("s")
    wid = s * NC + c

    pltpu.sync_copy(h2d.at[pl.ds(NCH * wid, NCH)], hidx)
    pltpu.sync_copy(r2d.at[pl.ds(NCH * wid, NCH)], ridx)
    pltpu.sync_copy(t2d.at[pl.ds(NCH * wid, NCH)], tidx)

    hv = (hv0, hv1)
    rv = (rv0, rv1)
    tv = (tv0, tv1)
    sems = ((sh0, sr0, st0), (sh1, sr1, st1))

    iota = lax.iota(jnp.int32, L)

    def fire(j):
        slot = j % 2

        def group_dma(g, carry, j=j, slot=slot):
            hvec = hidx[j, pl.ds(g * L, L)]
            rvec = ridx[j, pl.ds(g * L, L)]
            tvec = tidx[j, pl.ds(g * L, L)]
            for lane in range(L):
                m = iota == lane
                hid = lax.reduce_max(jnp.where(m, hvec, -1), axes=(0,))
                rid = lax.reduce_max(jnp.where(m, rvec, -1), axes=(0,))
                tid = lax.reduce_max(jnp.where(m, tvec, -1), axes=(0,))
                r = g * L + lane
                pltpu.make_async_copy(
                    ent.at[pl.ds(hid, 1)], hv[slot].at[pl.ds(r, 1)],
                    sems[slot][0]).start()
                pltpu.make_async_copy(
                    rel.at[pl.ds(rid, 1)], rv[slot].at[pl.ds(r, 1)],
                    sems[slot][1]).start()
                pltpu.make_async_copy(
                    ent.at[pl.ds(tid, 1)], tv[slot].at[pl.ds(r, 1)],
                    sems[slot][2]).start()
            return carry

        lax.fori_loop(0, NG, group_dma, 0)

    def drain(j):
        slot = j % 2
        pltpu.make_async_copy(
            ent.at[pl.ds(0, CH)], hv[slot], sems[slot][0]).wait()
        pltpu.make_async_copy(
            ent.at[pl.ds(0, CH)], rv[slot], sems[slot][1]).wait()
        pltpu.make_async_copy(
            ent.at[pl.ds(0, CH)], tv[slot], sems[slot][2]).wait()

    fire(0)
    fire(1)

    for j in range(NCH):
        slot = j % 2
        drain(j)

        def row_body(r, carry, slot=slot):
            acc = None
            for k in range(D // L):
                hh = hv[slot][r, pl.ds(k * L, L)]
                re = rv[slot][r, pl.ds(k * L, L)]
                tt = tv[slot][r, pl.ds(k * L, L)]
                df = (hh + re) - tt
                acc = df * df if acc is None else acc + df * df
            ps[r] = acc
            return carry

        lax.fori_loop(0, CH, row_body, 0, unroll=4)

        for g in range(NG):
            rows16 = g * L + iota
            acc = jnp.zeros((L,), jnp.float32)
            for k in range(L):
                col = jnp.full((L,), k, jnp.int32)
                acc = acc + plsc.load_gather(ps, [rows16, col])
            sc2[j, pl.ds(g * L, L)] = _sqrt16(acc)

        if j + 2 < NCH:
            fire(j + 2)

    pltpu.sync_copy(sc2, out.at[pl.ds(NCH * wid, NCH)])


@jax.jit
def kernel(head_ids, relation_ids, tail_ids, entity_table, relation_table):
    h2d = head_ids.astype(jnp.int32).reshape(NW * NCH, CH)
    r2d = relation_ids.astype(jnp.int32).reshape(NW * NCH, CH)
    t2d = tail_ids.astype(jnp.int32).reshape(NW * NCH, CH)

    mesh = plsc.VectorSubcoreMesh(core_axis_name="c", subcore_axis_name="s")
    scratch = [
        pltpu.VMEM((NCH, CH), jnp.int32),
        pltpu.VMEM((NCH, CH), jnp.int32),
        pltpu.VMEM((NCH, CH), jnp.int32),
        pltpu.VMEM((CH, D), jnp.float32),
        pltpu.VMEM((CH, D), jnp.float32),
        pltpu.VMEM((CH, D), jnp.float32),
        pltpu.VMEM((CH, D), jnp.float32),
        pltpu.VMEM((CH, D), jnp.float32),
        pltpu.VMEM((CH, D), jnp.float32),
        pltpu.VMEM((CH, L), jnp.float32),
        pltpu.VMEM((NCH, CH), jnp.float32),
    ] + [pltpu.SemaphoreType.DMA] * 6

    run = pl.kernel(
        _sc_body,
        out_type=jax.ShapeDtypeStruct((NW * NCH, CH), jnp.float32),
        mesh=mesh,
        scratch_types=scratch,
        compiler_params=pltpu.CompilerParams(
            needs_layout_passes=False, use_tc_tiling_on_sc=True),
    )
    out = run(h2d, r2d, t2d, entity_table, relation_table)
    return out.reshape(B)
